# baseline (speedup 1.0000x reference)
import functools

import jax
import jax.numpy as jnp
from jax import lax
from jax.experimental import pallas as pl
from jax.experimental.pallas import tpu as pltpu
from jax.experimental.pallas import tpu_sc as plsc

B = 16384
L = 200
E = 32
C = 100
IL = L // 2

NC, NS = 2, 16
NW = NC * NS
ROWS_W = B // NW
CB = 64
IR = 2 * CB
NCHUNK = ROWS_W // CB

_RPI = 4


def _reduce_buf(rows_ref, a0, a1, lo):
    z = jnp.zeros((16,), jnp.float32)

    @plsc.parallel_loop(lo, lo + IL, _RPI, unroll=2,
                        carry=(a0, a1, z, z, z, z, z, z))
    def accs(j, carry):
        c = list(carry)
        for u in range(_RPI):
            c[2 * u] = c[2 * u] + rows_ref[j + u, 0:16]
            c[2 * u + 1] = c[2 * u + 1] + rows_ref[j + u, 16:32]
        return tuple(c)

    return (accs[0] + accs[2] + accs[4] + accs[6],
            accs[1] + accs[3] + accs[5] + accs[7])


_NBUF = 16


def _pool_body(x_hbm, tab_hbm, out_hbm, idx_v, rows, pooled_v,
               sem_idx, sem_g, sem_out):
    c = lax.axis_index("c")
    s = lax.axis_index("s")
    wid = s * NC + c
    row0 = wid * ROWS_W

    def fire_idx(ci, par):
        rbase = row0 + ci * CB
        for h in range(2):
            pltpu.async_copy(
                x_hbm.at[pl.ds(rbase, CB), pl.ds(96 * h, IL + 4)],
                idx_v[2 * par + h], sem_idx[par])

    def fire_gather(ci_par, r, h, b):
        pltpu.async_copy(
            tab_hbm.at[idx_v[2 * ci_par + h].at[r]], rows[b], sem_g[b])

    def wait_gather(ci_par, b):
        pltpu.make_async_copy(
            tab_hbm.at[idx_v[2 * ci_par].at[0]], rows[b], sem_g[b]).wait()

    fire_idx(0, 0)

    def chunk(ci, p):
        rbase = row0 + ci * CB
        for h in range(2):
            pltpu.make_async_copy(
                x_hbm.at[pl.ds(0, CB), pl.ds(0, IL + 4)], idx_v[2 * p + h],
                sem_idx[p]).wait()

        @pl.when(ci < NCHUNK - 1)
        def _():
            fire_idx(ci + 1, 1 - p)

        @pl.when(ci >= 2)
        def _():
            pltpu.make_async_copy(
                pooled_v[p], out_hbm.at[pl.ds(0, CB)], sem_out[p]).wait()

        for b in range(_NBUF):
            fire_gather(p, b // 2, b % 2, b)

        nr = _NBUF // 2

        def grp(q, _):
            for rr in range(nr):
                z = jnp.zeros((16,), jnp.float32)
                a0, a1 = z, z
                r = nr * q + rr
                for h in range(2):
                    b = 2 * rr + h
                    wait_gather(p, b)
                    a0, a1 = _reduce_buf(rows[b], a0, a1, 4 * h)

                    @pl.when(r + nr < CB)
                    def _():
                        fire_gather(p, r + nr, h, b)

                pooled_v[p][r, 0:16] = a0
                pooled_v[p][r, 16:32] = a1
            return ()

        lax.fori_loop(0, CB // nr, grp, ())
        pltpu.async_copy(pooled_v[p], out_hbm.at[pl.ds(rbase, CB)], sem_out[p])

    def chunk2(cc, _):
        for p in range(2):
            chunk(2 * cc + p, p)
        return ()

    lax.fori_loop(0, NCHUNK // 2, chunk2, ())

    for p in range(2):
        pltpu.make_async_copy(
            pooled_v[p], out_hbm.at[pl.ds(0, CB)], sem_out[p]).wait()


@functools.partial(
    pl.kernel,
    out_type=jax.ShapeDtypeStruct((B, E), jnp.float32),
    mesh=plsc.VectorSubcoreMesh(
        core_axis_name="c", subcore_axis_name="s",
        num_cores=NC, num_subcores=NS),
    scratch_types=[
        [pltpu.VMEM((CB, IL + 4), jnp.int32) for _ in range(4)],
        [pltpu.VMEM((IL + 4, E), jnp.float32) for _ in range(_NBUF)],
        [pltpu.VMEM((CB, E), jnp.float32) for _ in range(2)],
        [pltpu.SemaphoreType.DMA for _ in range(2)],
        [pltpu.SemaphoreType.DMA for _ in range(_NBUF)],
        [pltpu.SemaphoreType.DMA for _ in range(2)],
    ],
    compiler_params=pltpu.CompilerParams(use_tc_tiling_on_sc=False),
)
def _pool(x_hbm, tab_hbm, out_hbm, idx_v, rows, pooled_v,
          sem_idx, sem_g, sem_out):
    _pool_body(x_hbm, tab_hbm, out_hbm, idx_v, rows, pooled_v,
               sem_idx, sem_g, sem_out)


def _linear_body(p_ref, w_ref, b_ref, o_ref):
    o_ref[...] = (
        jnp.dot(p_ref[...], w_ref[...], preferred_element_type=jnp.float32)
        * (1.0 / L) + b_ref[...]
    )


def _linear(pooled, wt, b2):
    bb = 1024
    return pl.pallas_call(
        _linear_body,
        grid=(B // bb,),
        in_specs=[
            pl.BlockSpec((bb, E), lambda i: (i, 0)),
            pl.BlockSpec((E, C), lambda i: (0, 0)),
            pl.BlockSpec((1, C), lambda i: (0, 0)),
        ],
        out_specs=pl.BlockSpec((bb, C), lambda i: (i, 0)),
        out_shape=jax.ShapeDtypeStruct((B, C), jnp.float32),
    )(pooled, wt, b2)


def kernel(x, emb_table, fc_w, fc_b):
    pooled_sum = _pool(x, emb_table)
    return _linear(pooled_sum, fc_w.T, fc_b.reshape(1, C))


# speedup vs baseline: 1.0239x; 1.0239x over previous
import functools

import jax
import jax.numpy as jnp
from jax import lax
from jax.experimental import pallas as pl
from jax.experimental.pallas import tpu as pltpu
from jax.experimental.pallas import tpu_sc as plsc

B = 16384
L = 200
E = 32
C = 100
IL = L // 2

NC, NS = 2, 16
NW = NC * NS
ROWS_W = B // NW
CB = 128
IR = 2 * CB
NCHUNK = ROWS_W // CB

_RPI = 4


def _reduce_buf(rows_ref, a0, a1, lo):
    z = jnp.zeros((16,), jnp.float32)

    @plsc.parallel_loop(lo, lo + IL, _RPI, unroll=2,
                        carry=(a0, a1, z, z, z, z, z, z))
    def accs(j, carry):
        c = list(carry)
        for u in range(_RPI):
            c[2 * u] = c[2 * u] + rows_ref[j + u, 0:16]
            c[2 * u + 1] = c[2 * u + 1] + rows_ref[j + u, 16:32]
        return tuple(c)

    return (accs[0] + accs[2] + accs[4] + accs[6],
            accs[1] + accs[3] + accs[5] + accs[7])


_NBUF = 8


def _pool_body(x_hbm, tab_hbm, out_hbm, idx_v, rows, pooled_v,
               sem_idx, sem_g, sem_out):
    c = lax.axis_index("c")
    s = lax.axis_index("s")
    wid = s * NC + c
    row0 = wid * ROWS_W

    def fire_idx(ci, par):
        rbase = row0 + ci * CB
        for h in range(2):
            pltpu.async_copy(
                x_hbm.at[pl.ds(rbase, CB), pl.ds(96 * h, IL + 4)],
                idx_v[2 * par + h], sem_idx[par])

    def fire_gather(ci_par, r, h, b):
        pltpu.async_copy(
            tab_hbm.at[idx_v[2 * ci_par + h].at[r]], rows[b], sem_g[b])

    def wait_gather(ci_par, b):
        pltpu.make_async_copy(
            tab_hbm.at[idx_v[2 * ci_par].at[0]], rows[b], sem_g[b]).wait()

    fire_idx(0, 0)

    def chunk(ci, p):
        rbase = row0 + ci * CB
        for h in range(2):
            pltpu.make_async_copy(
                x_hbm.at[pl.ds(0, CB), pl.ds(0, IL + 4)], idx_v[2 * p + h],
                sem_idx[p]).wait()

        @pl.when(ci < NCHUNK - 1)
        def _():
            fire_idx(ci + 1, 1 - p)

        @pl.when(ci >= 2)
        def _():
            pltpu.make_async_copy(
                pooled_v[p], out_hbm.at[pl.ds(0, CB)], sem_out[p]).wait()

        for b in range(_NBUF):
            fire_gather(p, b // 2, b % 2, b)

        nr = _NBUF // 2

        def grp(q, _):
            for rr in range(nr):
                z = jnp.zeros((16,), jnp.float32)
                a0, a1 = z, z
                r = nr * q + rr
                for h in range(2):
                    b = 2 * rr + h
                    wait_gather(p, b)
                    a0, a1 = _reduce_buf(rows[b], a0, a1, 4 * h)

                    @pl.when(r + nr < CB)
                    def _():
                        fire_gather(p, r + nr, h, b)

                pooled_v[p][r, 0:16] = a0
                pooled_v[p][r, 16:32] = a1
            return ()

        lax.fori_loop(0, CB // nr, grp, ())
        pltpu.async_copy(pooled_v[p], out_hbm.at[pl.ds(rbase, CB)], sem_out[p])

    def chunk2(cc, _):
        for p in range(2):
            chunk(2 * cc + p, p)
        return ()

    lax.fori_loop(0, NCHUNK // 2, chunk2, ())

    for p in range(2):
        pltpu.make_async_copy(
            pooled_v[p], out_hbm.at[pl.ds(0, CB)], sem_out[p]).wait()


@functools.partial(
    pl.kernel,
    out_type=jax.ShapeDtypeStruct((B, E), jnp.float32),
    mesh=plsc.VectorSubcoreMesh(
        core_axis_name="c", subcore_axis_name="s",
        num_cores=NC, num_subcores=NS),
    scratch_types=[
        [pltpu.VMEM((CB, IL + 4), jnp.int32) for _ in range(4)],
        [pltpu.VMEM((IL + 4, E), jnp.float32) for _ in range(_NBUF)],
        [pltpu.VMEM((CB, E), jnp.float32) for _ in range(2)],
        [pltpu.SemaphoreType.DMA for _ in range(2)],
        [pltpu.SemaphoreType.DMA for _ in range(_NBUF)],
        [pltpu.SemaphoreType.DMA for _ in range(2)],
    ],
    compiler_params=pltpu.CompilerParams(use_tc_tiling_on_sc=False),
)
def _pool(x_hbm, tab_hbm, out_hbm, idx_v, rows, pooled_v,
          sem_idx, sem_g, sem_out):
    _pool_body(x_hbm, tab_hbm, out_hbm, idx_v, rows, pooled_v,
               sem_idx, sem_g, sem_out)


def _linear_body(p_ref, w_ref, b_ref, o_ref):
    o_ref[...] = (
        jnp.dot(p_ref[...], w_ref[...], preferred_element_type=jnp.float32)
        * (1.0 / L) + b_ref[...]
    )


def _linear(pooled, wt, b2):
    bb = 1024
    return pl.pallas_call(
        _linear_body,
        grid=(B // bb,),
        in_specs=[
            pl.BlockSpec((bb, E), lambda i: (i, 0)),
            pl.BlockSpec((E, C), lambda i: (0, 0)),
            pl.BlockSpec((1, C), lambda i: (0, 0)),
        ],
        out_specs=pl.BlockSpec((bb, C), lambda i: (i, 0)),
        out_shape=jax.ShapeDtypeStruct((B, C), jnp.float32),
    )(pooled, wt, b2)


def kernel(x, emb_table, fc_w, fc_b):
    pooled_sum = _pool(x, emb_table)
    return _linear(pooled_sum, fc_w.T, fc_b.reshape(1, C))
